# baseline (speedup 1.0000x reference)
import jax
import jax.numpy as jnp
from jax.experimental import pallas as pl
from jax.experimental.pallas import tpu as pltpu

N_USERS = 50000
N_ITEMS = 20000
N_ENTITIES = 50000
N_EDGES = 800000
N_REL = 32
N_INTER = 500000
D = 64
GAMMA = 0.5
MAX_ITER = 2

BE = 8000
BI = 5000
BN = 1000
SROWS = 1568

_f32 = jnp.float32


def _smem_idx_spec(blk):
    return pl.BlockSpec((1, 1, blk), lambda i: (i, 0, 0),
                        memory_space=pltpu.SMEM)


def _full_vmem(shape):
    return pl.BlockSpec(shape, lambda i: tuple(0 for _ in shape))


def _kg1_body(head_ref, tail_ref, et_ref, ent_ref, rel_ref, acc_ref):
    @pl.when(pl.program_id(0) == 0)
    def _init():
        acc_ref[...] = jnp.zeros_like(acc_ref)

    def body(e, carry):
        h = head_ref[0, 0, e]
        t = tail_ref[0, 0, e]
        r = et_ref[0, 0, e]
        ent = ent_ref[pl.ds(t, 1), :]
        er = rel_ref[pl.ds(r, 1), :]
        cfb = (t < N_ITEMS) != (h < N_ITEMS)
        cf = jnp.where(cfb, 1.0, 0.0).astype(_f32)
        sf = 1.0 - cf
        row = jnp.concatenate([ent * er * cf, (ent + er) * sf], axis=1)
        acc_ref[pl.ds(h, 1), :] += row
        return carry

    jax.lax.fori_loop(0, BE, body, 0)


def _kg_sums(head2, tail2, et2, entity_emb, relation_weight):
    return pl.pallas_call(
        _kg1_body,
        grid=(N_EDGES // BE,),
        in_specs=[
            _smem_idx_spec(BE), _smem_idx_spec(BE), _smem_idx_spec(BE),
            _full_vmem((N_ENTITIES, D)),
            _full_vmem((N_REL, D)),
        ],
        out_specs=_full_vmem((N_ENTITIES, 128)),
        out_shape=jax.ShapeDtypeStruct((N_ENTITIES, 128), _f32),
    )(head2, tail2, et2, entity_emb, relation_weight)


def _kg2_body(head_ref, tail_ref, et_ref, rel_ref, acc_ref):
    @pl.when(pl.program_id(0) == 0)
    def _init():
        acc_ref[...] = jnp.zeros_like(acc_ref)

    iota = jax.lax.broadcasted_iota(jnp.int32, (1, 64), 1)

    def body(e, carry):
        h = head_ref[0, 0, e]
        t = tail_ref[0, 0, e]
        r = et_ref[0, 0, e]
        er = rel_ref[pl.ds(r, 1), :]
        cfb = (t < N_ITEMS) != (h < N_ITEMS)
        cf = jnp.where(cfb, 1.0, 0.0).astype(_f32)
        sf = 1.0 - cf
        cnt = jnp.where(iota == 0, cf,
                        jnp.where(iota == 1, sf,
                                  jnp.where(iota == 2, 1.0, 0.0))).astype(_f32)
        row = jnp.concatenate([er, cnt], axis=1)
        acc_ref[pl.ds(h, 1), :] += row
        return carry

    jax.lax.fori_loop(0, BE, body, 0)


def _kg_rel_counts(head2, tail2, et2, relation_weight):
    return pl.pallas_call(
        _kg2_body,
        grid=(N_EDGES // BE,),
        in_specs=[
            _smem_idx_spec(BE), _smem_idx_spec(BE), _smem_idx_spec(BE),
            _full_vmem((N_REL, D)),
        ],
        out_specs=_full_vmem((N_ENTITIES, 128)),
        out_shape=jax.ShapeDtypeStruct((N_ENTITIES, 128), _f32),
    )(head2, tail2, et2, relation_weight)


def _ent_body(a1_ref, a2_ref, w1t_ref, b1_ref, w2t_ref, b2_ref,
              out_ref, rel_ref):
    s12 = a1_ref[...]
    rc = a2_ref[...]
    cc = jnp.maximum(rc[:, 64:65], 1.0)
    cs = jnp.maximum(rc[:, 65:66], 1.0)
    ca = jnp.maximum(rc[:, 66:67], 1.0)
    a1 = s12[:, 0:64] / cc
    a2 = s12[:, 64:128] / cs
    h1 = jnp.dot(a1, w1t_ref[...], preferred_element_type=_f32) + b1_ref[...]
    h2 = jnp.dot(a2, w2t_ref[...], preferred_element_type=_f32) + b2_ref[...]
    l1 = jnp.where(h1 >= 0, h1, 0.01 * h1) * 0.5
    l2 = jnp.where(h2 >= 0, h2, 0.01 * h2) * 0.5
    out_ref[...] = l1 + l2
    rel_ref[...] = rc[:, 0:64] / ca


def _ent_dense(acc1, acc2, W1_w, W1_b, W2_w, W2_b):
    return pl.pallas_call(
        _ent_body,
        grid=(N_ENTITIES // BN,),
        in_specs=[
            pl.BlockSpec((BN, 128), lambda i: (i, 0)),
            pl.BlockSpec((BN, 128), lambda i: (i, 0)),
            _full_vmem((D, D)), _full_vmem((1, D)),
            _full_vmem((D, D)), _full_vmem((1, D)),
        ],
        out_specs=[
            pl.BlockSpec((BN, D), lambda i: (i, 0)),
            pl.BlockSpec((BN, D), lambda i: (i, 0)),
        ],
        out_shape=[
            jax.ShapeDtypeStruct((N_ENTITIES, D), _f32),
            jax.ShapeDtypeStruct((N_ENTITIES, D), _f32),
        ],
    )(acc1, acc2, W1_w.T, W1_b.reshape(1, D), W2_w.T, W2_b.reshape(1, D))


def _pack_body(rel_ref, ikg_ref, icf_ref, out_ref):
    out_ref[...] = jnp.concatenate(
        [rel_ref[...] * ikg_ref[...], icf_ref[...]], axis=1)


def _pack_items(rel_i, item_kg, item_cf):
    return pl.pallas_call(
        _pack_body,
        grid=(N_ITEMS // BN,),
        in_specs=[pl.BlockSpec((BN, D), lambda i: (i, 0))] * 3,
        out_specs=pl.BlockSpec((BN, 128), lambda i: (i, 0)),
        out_shape=jax.ShapeDtypeStruct((N_ITEMS, 128), _f32),
    )(rel_i, item_kg, item_cf)


def _item_row(it_ref, im):
    row = it_ref[pl.ds(im // 2, 1), :]
    return jnp.where(im % 2 == 0, row[:, 0:128], row[:, 128:256])


def _den_body(mr_ref, mc_ref, u_ref, it_ref, s_ref):
    @pl.when(pl.program_id(0) == 0)
    def _init():
        s_ref[...] = jnp.zeros_like(s_ref)

    iota = jax.lax.broadcasted_iota(jnp.int32, (1, 128), 1)

    def body(e, carry):
        um = mr_ref[0, 0, e]
        im = mc_ref[0, 0, e]
        urow = u_ref[pl.ds(um, 1), :]
        irow = _item_row(it_ref, im)
        p = jnp.sum(urow[:, 0:64] * irow[:, 0:64], axis=1, keepdims=True)
        pcf = jnp.sum(urow[:, 64:128] * irow[:, 64:128], axis=1, keepdims=True)
        ep = jnp.exp(jax.nn.sigmoid(p))
        epcf = jnp.exp(jax.nn.sigmoid(pcf))
        lb = (um % 32) * 2
        svec = jnp.where(iota == lb, ep, jnp.where(iota == lb + 1, epcf, 0.0))
        s_ref[pl.ds(um // 32, 1), :] += svec.astype(_f32)
        return carry

    jax.lax.fori_loop(0, BI, body, 0)


def _den_pass(mr2, mc2, U, ITpd):
    return pl.pallas_call(
        _den_body,
        grid=(N_INTER // BI,),
        in_specs=[
            _smem_idx_spec(BI), _smem_idx_spec(BI),
            _full_vmem((N_USERS, 128)),
            _full_vmem((N_ITEMS // 2, 256)),
        ],
        out_specs=_full_vmem((SROWS, 128)),
        out_shape=jax.ShapeDtypeStruct((SROWS, 128), _f32),
    )(mr2, mc2, U, ITpd)


def _wgt_body(mr_ref, mc_ref, u_ref, it_ref, s_ref,
              wp_ref, wpcf_ref, mask_ref):
    iota = jax.lax.broadcasted_iota(jnp.int32, (1, 128), 1)

    def body(e, carry):
        um = mr_ref[0, 0, e]
        im = mc_ref[0, 0, e]
        urow = u_ref[pl.ds(um, 1), :]
        irow = _item_row(it_ref, im)
        p = jnp.sum(urow[:, 0:64] * irow[:, 0:64], axis=1, keepdims=True)
        pcf = jnp.sum(urow[:, 64:128] * irow[:, 64:128], axis=1, keepdims=True)
        srow = s_ref[pl.ds(um // 32, 1), :]
        lb = (um % 32) * 2
        sp = jnp.sum(jnp.where(iota == lb, srow, 0.0), axis=1, keepdims=True)
        spcf = jnp.sum(jnp.where(iota == lb + 1, srow, 0.0),
                       axis=1, keepdims=True)
        pn = jnp.exp(jax.nn.sigmoid(p)) / sp
        pcfn = jnp.exp(jax.nn.sigmoid(pcf)) / spcf
        m = jnp.abs(jax.nn.sigmoid(pn) - jax.nn.sigmoid(pcfn)) < GAMMA
        mf = jnp.where(m, 1.0, 0.0).astype(_f32)
        wp_ref[pl.ds(e, 1), :] = pn * mf
        wpcf_ref[pl.ds(e, 1), :] = pcfn * mf
        mask_ref[pl.ds(e, 1), :] = m.astype(jnp.int32)
        return carry

    jax.lax.fori_loop(0, BI, body, 0)


def _wgt_pass(mr2, mc2, U, ITpd, S):
    return pl.pallas_call(
        _wgt_body,
        grid=(N_INTER // BI,),
        in_specs=[
            _smem_idx_spec(BI), _smem_idx_spec(BI),
            _full_vmem((N_USERS, 128)),
            _full_vmem((N_ITEMS // 2, 256)),
            _full_vmem((SROWS, 128)),
        ],
        out_specs=[
            pl.BlockSpec((BI, 1), lambda i: (i, 0)),
            pl.BlockSpec((BI, 1), lambda i: (i, 0)),
            pl.BlockSpec((BI, 1), lambda i: (i, 0)),
        ],
        out_shape=[
            jax.ShapeDtypeStruct((N_INTER, 1), _f32),
            jax.ShapeDtypeStruct((N_INTER, 1), _f32),
            jax.ShapeDtypeStruct((N_INTER, 1), jnp.int32),
        ],
    )(mr2, mc2, U, ITpd, S)


def _scat_body(mr_ref, mc_ref, wp_ref, wpcf_ref, it_ref, un_ref):
    @pl.when(pl.program_id(0) == 0)
    def _init():
        un_ref[...] = jnp.zeros_like(un_ref)

    def body(e, carry):
        um = mr_ref[0, 0, e]
        im = mc_ref[0, 0, e]
        wp = wp_ref[0, 0, e]
        wpcf = wpcf_ref[0, 0, e]
        irow = _item_row(it_ref, im)
        row = jnp.concatenate(
            [irow[:, 0:64] * wp, irow[:, 64:128] * wpcf], axis=1)
        un_ref[pl.ds(um, 1), :] += row
        return carry

    jax.lax.fori_loop(0, BI, body, 0)


def _scat_pass(mr2, mc2, wp3, wpcf3, ITc):
    return pl.pallas_call(
        _scat_body,
        grid=(N_INTER // BI,),
        in_specs=[
            _smem_idx_spec(BI), _smem_idx_spec(BI),
            _smem_idx_spec(BI), _smem_idx_spec(BI),
            _full_vmem((N_ITEMS // 2, 256)),
        ],
        out_specs=_full_vmem((N_USERS, 128)),
        out_shape=jax.ShapeDtypeStruct((N_USERS, 128), _f32),
    )(mr2, mc2, wp3, wpcf3, ITc)


def _norm_body(x_ref, o_ref):
    x = x_ref[...]
    u = x[:, 0:64]
    v = x[:, 64:128]
    nu = jnp.maximum(jnp.sqrt(jnp.sum(u * u, axis=1, keepdims=True)), 1e-12)
    nv = jnp.maximum(jnp.sqrt(jnp.sum(v * v, axis=1, keepdims=True)), 1e-12)
    o_ref[...] = jnp.concatenate([u / nu, v / nv], axis=1)


def _l2norm2(U):
    return pl.pallas_call(
        _norm_body,
        grid=(N_USERS // BN,),
        in_specs=[pl.BlockSpec((BN, 128), lambda i: (i, 0))],
        out_specs=pl.BlockSpec((BN, 128), lambda i: (i, 0)),
        out_shape=jax.ShapeDtypeStruct((N_USERS, 128), _f32),
    )(U)


def _item_body(mr_ref, mc_ref, ucf_ref, acc_ref):
    @pl.when(pl.program_id(0) == 0)
    def _init():
        acc_ref[...] = jnp.zeros_like(acc_ref)

    iota = jax.lax.broadcasted_iota(jnp.int32, (1, 64), 1)
    one = jnp.where(iota == 0, 1.0, 0.0).astype(_f32)

    def body(e, carry):
        um = mr_ref[0, 0, e]
        im = mc_ref[0, 0, e]
        urow = ucf_ref[pl.ds(um, 1), :]
        acc_ref[pl.ds(im, 1), :] += jnp.concatenate([urow, one], axis=1)
        return carry

    jax.lax.fori_loop(0, BI, body, 0)


def _item_acc(mr2, mc2, user_emb_cf):
    return pl.pallas_call(
        _item_body,
        grid=(N_INTER // BI,),
        in_specs=[
            _smem_idx_spec(BI), _smem_idx_spec(BI),
            _full_vmem((N_USERS, D)),
        ],
        out_specs=_full_vmem((N_ITEMS, 128)),
        out_shape=jax.ShapeDtypeStruct((N_ITEMS, 128), _f32),
    )(mr2, mc2, user_emb_cf)


def _div_body(acc_ref, o_ref):
    acc = acc_ref[...]
    o_ref[...] = acc[:, 0:64] / jnp.maximum(acc[:, 64:65], 1.0)


def _item_div(acc):
    return pl.pallas_call(
        _div_body,
        grid=(N_ITEMS // BN,),
        in_specs=[pl.BlockSpec((BN, 128), lambda i: (i, 0))],
        out_specs=pl.BlockSpec((BN, D), lambda i: (i, 0)),
        out_shape=jax.ShapeDtypeStruct((N_ITEMS, D), _f32),
    )(acc)


def kernel(entity_emb, user_emb, user_emb_cf, item_emb_cf, edge_index,
           edge_type, interact_mat, relation_weight, W1_w, W1_b, W2_w, W2_b):
    head2 = edge_index[0].reshape(N_EDGES // BE, 1, BE)
    tail2 = edge_index[1].reshape(N_EDGES // BE, 1, BE)
    et2 = edge_type.reshape(N_EDGES // BE, 1, BE)
    mr2 = interact_mat[:, 0].reshape(N_INTER // BI, 1, BI)
    mc2 = interact_mat[:, 1].reshape(N_INTER // BI, 1, BI)

    acc1 = _kg_sums(head2, tail2, et2, entity_emb, relation_weight)
    acc2 = _kg_rel_counts(head2, tail2, et2, relation_weight)
    entity_agg, rel_full = _ent_dense(acc1, acc2, W1_w, W1_b, W2_w, W2_b)

    item_kg = entity_emb[:N_ITEMS]
    ITpd = _pack_items(rel_full[:N_ITEMS], item_kg, item_emb_cf)
    ITpd = ITpd.reshape(N_ITEMS // 2, 256)
    ITc = jnp.concatenate([item_kg, item_emb_cf], axis=1)
    ITc = ITc.reshape(N_ITEMS // 2, 256)

    U = jnp.concatenate([user_emb, user_emb_cf], axis=1)
    mask = None
    for i in range(MAX_ITER):
        S = _den_pass(mr2, mc2, U, ITpd)
        wp, wpcf, mask = _wgt_pass(mr2, mc2, U, ITpd, S)
        wp3 = wp.reshape(N_INTER // BI, 1, BI)
        wpcf3 = wpcf.reshape(N_INTER // BI, 1, BI)
        Unew = _scat_pass(mr2, mc2, wp3, wpcf3, ITc)
        if i < MAX_ITER - 1:
            U = _l2norm2(Unew)
        else:
            U = Unew

    item_agg = _item_div(_item_acc(mr2, mc2, user_emb_cf))

    return (entity_agg, U[:, 0:64], U[:, 64:128], item_agg, mask)
